# baseline (speedup 1.0000x reference)
import functools

import jax
import jax.numpy as jnp
from jax import lax
from jax.experimental import pallas as pl
from jax.experimental.pallas import tpu as pltpu
from jax.experimental.pallas import tpu_sc as plsc

K = 15
KPAD = 16
PD = 8
IN_DIM = 128
OUT_DIM = 128
N_NODES = 10000
N_PAD = 10112
E_EDGES = 160000
E_PAD = 163840
KP_EXTENT = 1.2

NC = 2
NS = 16
NW = NC * NS
L = 16
CHUNK = 128
GROUPS = CHUNK // L
EPW = E_PAD // NW
CHUNKS = EPW // CHUNK
ROWS_PER_TILE = N_PAD // NS


STEP = 160
HALF = STEP // 2
NSTEP = EPW // STEP
HGROUPS = HALF // L


def _gather_body(src_hbm, dst_hbm, feat_hbm, posf_hbm,
                 fsrc_hbm, y_hbm,
                 siall, diall, frows, yv, pf,
                 sg0, sg1, sw0, sw1):
    c = lax.axis_index("c")
    s = lax.axis_index("s")
    wid = s * NC + c
    base_w = wid * EPW
    sg = (sg0, sg1)
    sw = (sw0, sw1)
    pltpu.sync_copy(posf_hbm, pf)
    pltpu.sync_copy(src_hbm.at[pl.ds(base_w, EPW)], siall)
    pltpu.sync_copy(dst_hbm.at[pl.ds(base_w, EPW)], diall)
    lanes = lax.iota(jnp.int32, L)

    def issue_gather(t, b):
        for j in range(2):
            pltpu.async_copy(
                feat_hbm.at[siall.at[pl.ds(t * STEP + j * HALF, HALF)]],
                frows.at[b, pl.ds(j * HALF, HALF)], sg[b])

    def wait_gather(b):
        pltpu.make_async_copy(feat_hbm.at[pl.ds(0, STEP)], frows.at[b],
                              sg[b]).wait()

    def issue_wout(t, b):
        base = base_w + t * STEP
        pltpu.async_copy(frows.at[b], fsrc_hbm.at[pl.ds(base, STEP)], sw[b])
        pltpu.async_copy(yv.at[b], y_hbm.at[pl.ds(base, STEP)], sw[b])

    def wait_wout(b):
        pltpu.make_async_copy(frows.at[b], fsrc_hbm.at[pl.ds(0, STEP)],
                              sw[b]).wait()
        pltpu.make_async_copy(yv.at[b], y_hbm.at[pl.ds(0, STEP)], sw[b]).wait()

    def compute_y(t, b):
        for j in range(2):
            for g in range(HGROUPS):
                off = t * STEP + j * HALF + g * L
                ivs = siall[pl.ds(off, L)]
                ivd = diall[pl.ds(off, L)]
                yx = plsc.load_gather(pf, [ivs]) - plsc.load_gather(pf, [ivd])
                ivs = ivs + N_NODES
                ivd = ivd + N_NODES
                yy = plsc.load_gather(pf, [ivs]) - plsc.load_gather(pf, [ivd])
                ivs = ivs + N_NODES
                ivd = ivd + N_NODES
                yz = plsc.load_gather(pf, [ivs]) - plsc.load_gather(pf, [ivd])
                rows = lanes + (j * HALF + g * L)
                plsc.store_scatter(yv.at[b],
                                   [rows, jnp.zeros((L,), jnp.int32)], yx)
                plsc.store_scatter(yv.at[b],
                                   [rows, jnp.ones((L,), jnp.int32)], yy)
                plsc.store_scatter(yv.at[b],
                                   [rows, jnp.full((L,), 2, jnp.int32)], yz)

    issue_gather(0, 0)

    def body(o, carry):
        for b in range(2):
            t = 2 * o + b
            compute_y(t, b)
            wait_gather(b)
            issue_wout(t, b)

            @pl.when(jnp.logical_and(t >= 1, t + 1 < NSTEP))
            def _():
                wait_wout(1 - b)

            @pl.when(t + 1 < NSTEP)
            def _():
                issue_gather(t + 1, 1 - b)
        return carry

    lax.fori_loop(0, NSTEP // 2, body, 0)
    wait_wout(0)
    wait_wout(1)


def _scatter_body(dst_hbm, msg_hbm, zeros_hbm, out_hbm, didx, mrows, acc,
                  sl0, sl1, ss0, ss1):
    c = lax.axis_index("c")
    s = lax.axis_index("s")
    wid = s * NC + c
    base_w = wid * EPW
    sl = (sl0, sl1)
    ss = (ss0, ss1)
    r0 = s * ROWS_PER_TILE
    pltpu.sync_copy(zeros_hbm.at[pl.ds(r0, ROWS_PER_TILE)],
                    acc.at[pl.ds(r0, ROWS_PER_TILE)])
    plsc.subcore_barrier()

    def issue_load(t, b):
        base = base_w + t * CHUNK
        pltpu.async_copy(dst_hbm.at[pl.ds(base, CHUNK)], didx.at[b], sl[b])
        pltpu.async_copy(msg_hbm.at[pl.ds(base, CHUNK)], mrows.at[b], sl[b])

    def wait_load(b):
        pltpu.make_async_copy(dst_hbm.at[pl.ds(0, CHUNK)], didx.at[b],
                              sl[b]).wait()
        pltpu.make_async_copy(msg_hbm.at[pl.ds(0, CHUNK)], mrows.at[b],
                              sl[b]).wait()

    def wait_scatter(b):
        pltpu.make_async_copy(msg_hbm.at[pl.ds(0, CHUNK)], mrows.at[b],
                              ss[b]).wait()

    issue_load(0, 0)

    def body(o, carry):
        for b in range(2):
            t = 2 * o + b
            wait_load(b)
            pltpu.async_copy(mrows.at[b], acc.at[didx.at[b]], ss[b], add=True)

            @pl.when(t >= 1)
            def _():
                wait_scatter(1 - b)

            @pl.when(t + 1 < CHUNKS)
            def _():
                issue_load(t + 1, 1 - b)
        return carry

    lax.fori_loop(0, CHUNKS // 2, body, 0)
    wait_scatter(1)
    plsc.subcore_barrier()
    pltpu.sync_copy(acc.at[pl.ds(r0, ROWS_PER_TILE)],
                    out_hbm.at[c, pl.ds(r0, ROWS_PER_TILE)])


BB = 1024


def _msg_body(fsrc_ref, y_ref, kpt_ref, wall_ref, sel_ref, msg_ref):
    col = lax.broadcasted_iota(jnp.int32, (BB, PD), 1)
    y = jnp.where(col < 3, y_ref[...], 0.0)
    kpt = kpt_ref[...]
    cross = jnp.dot(y, kpt, preferred_element_type=jnp.float32)
    yn2 = jnp.sum(y * y, axis=1, keepdims=True)
    kn2 = jnp.sum(kpt * kpt, axis=0, keepdims=True)
    d2 = jnp.maximum(yn2 + kn2 - 2.0 * cross, 0.0) + 1e-12
    h = jnp.maximum(1.0 - jnp.sqrt(d2) * (1.0 / KP_EXTENT), 0.0)
    f16 = fsrc_ref[...].astype(jnp.bfloat16)
    t = jnp.dot(f16, wall_ref[...], preferred_element_type=jnp.float32)
    hb = jnp.dot(h.astype(jnp.bfloat16), sel_ref[...],
                 preferred_element_type=jnp.float32)
    acc = hb[:, 0:OUT_DIM] * t[:, 0:OUT_DIM]
    for k in range(1, KPAD):
        sl = slice(k * OUT_DIM, (k + 1) * OUT_DIM)
        acc = acc + hb[:, sl] * t[:, sl]
    msg_ref[...] = acc


def _add_body(a_ref, b_ref, o_ref):
    o_ref[...] = a_ref[...] + b_ref[...]


@jax.jit
def kernel(feat, pos, edge_index, weights, kernel_points):
    src = edge_index[0]
    dst = edge_index[1]
    epad = E_PAD - E_EDGES
    src_p = jnp.concatenate([src, jnp.zeros((epad,), jnp.int32)])
    dst_p = jnp.concatenate([dst, jnp.full((epad,), N_NODES, jnp.int32)])
    posf = jnp.concatenate([pos[:, 0], pos[:, 1], pos[:, 2]])
    kpt = jnp.pad(kernel_points, ((0, KPAD - K), (0, PD - kernel_points.shape[1]))).T
    w_all = jnp.transpose(
        jnp.pad(weights, ((0, KPAD - K), (0, 0), (0, 0))), (1, 0, 2)
    ).reshape(IN_DIM, KPAD * OUT_DIM).astype(jnp.bfloat16)
    sel = jnp.repeat(jnp.eye(KPAD, dtype=jnp.float32), OUT_DIM,
                     axis=1).astype(jnp.bfloat16)

    mesh = plsc.VectorSubcoreMesh(core_axis_name="c", subcore_axis_name="s")

    gather_fn = pl.kernel(
        _gather_body,
        out_type=[
            jax.ShapeDtypeStruct((E_PAD, IN_DIM), jnp.float32),
            jax.ShapeDtypeStruct((E_PAD, PD), jnp.float32),
        ],
        mesh=mesh,
        compiler_params=pltpu.CompilerParams(needs_layout_passes=False),
        scratch_types=[
            pltpu.VMEM((EPW,), jnp.int32),
            pltpu.VMEM((EPW,), jnp.int32),
            pltpu.VMEM((2, STEP, IN_DIM), jnp.float32),
            pltpu.VMEM((2, STEP, PD), jnp.float32),
            pltpu.VMEM((3 * N_NODES,), jnp.float32),
            pltpu.SemaphoreType.DMA,
            pltpu.SemaphoreType.DMA,
            pltpu.SemaphoreType.DMA,
            pltpu.SemaphoreType.DMA,
        ],
    )
    fsrc, yarr = gather_fn(src_p, dst_p, feat, posf)

    msg = pl.pallas_call(
        _msg_body,
        grid=(E_PAD // BB,),
        in_specs=[
            pl.BlockSpec((BB, IN_DIM), lambda i: (i, 0)),
            pl.BlockSpec((BB, PD), lambda i: (i, 0)),
            pl.BlockSpec((PD, KPAD), lambda i: (0, 0)),
            pl.BlockSpec((IN_DIM, KPAD * OUT_DIM), lambda i: (0, 0)),
            pl.BlockSpec((KPAD, KPAD * OUT_DIM), lambda i: (0, 0)),
        ],
        out_specs=pl.BlockSpec((BB, OUT_DIM), lambda i: (i, 0)),
        out_shape=jax.ShapeDtypeStruct((E_PAD, OUT_DIM), jnp.float32),
    )(fsrc, yarr, kpt, w_all, sel)

    scatter_fn = pl.kernel(
        _scatter_body,
        out_type=jax.ShapeDtypeStruct((NC, N_PAD, OUT_DIM), jnp.float32),
        mesh=mesh,
        scratch_types=[
            pltpu.VMEM((2, CHUNK), jnp.int32),
            pltpu.VMEM((2, CHUNK, OUT_DIM), jnp.float32),
            pltpu.VMEM_SHARED((N_PAD, OUT_DIM), jnp.float32),
            pltpu.SemaphoreType.DMA,
            pltpu.SemaphoreType.DMA,
            pltpu.SemaphoreType.DMA,
            pltpu.SemaphoreType.DMA,
        ],
    )
    zeros_hbm = jnp.zeros((N_PAD, OUT_DIM), jnp.float32)
    partials = scatter_fn(dst_p, msg, zeros_hbm)

    out = pl.pallas_call(
        _add_body,
        grid=(10,),
        in_specs=[
            pl.BlockSpec((1000, OUT_DIM), lambda i: (i, 0)),
            pl.BlockSpec((1000, OUT_DIM), lambda i: (i, 0)),
        ],
        out_specs=pl.BlockSpec((1000, OUT_DIM), lambda i: (i, 0)),
        out_shape=jax.ShapeDtypeStruct((N_NODES, OUT_DIM), jnp.float32),
    )(partials[0, :N_NODES], partials[1, :N_NODES])
    return out


# speedup vs baseline: 1.1441x; 1.1441x over previous
import functools

import jax
import jax.numpy as jnp
from jax import lax
from jax.experimental import pallas as pl
from jax.experimental.pallas import tpu as pltpu
from jax.experimental.pallas import tpu_sc as plsc

K = 15
KPAD = 16
PD = 8
IN_DIM = 128
OUT_DIM = 128
N_NODES = 10000
N_PAD = 10112
E_EDGES = 160000
E_PAD = 163840
KP_EXTENT = 1.2

NC = 2
NS = 16
NW = NC * NS
L = 16
CHUNK = 128
GROUPS = CHUNK // L
EPW = E_PAD // NW
CHUNKS = EPW // CHUNK
ROWS_PER_TILE = N_PAD // NS


STEP = 160
HALF = STEP // 2
NSTEP = EPW // STEP
HGROUPS = HALF // L


def _make_gather_body(epw):
  nstep = epw // STEP

  def _gather_body(src_hbm, dst_hbm, feat_hbm, posf_hbm,
                   fsrc_hbm, y_hbm,
                   siall, diall, frows, yv, pf,
                   sg0, sg1, sw0, sw1):
    c = lax.axis_index("c")
    s = lax.axis_index("s")
    wid = s * NC + c
    base_w = wid * epw
    sg = (sg0, sg1)
    sw = (sw0, sw1)
    pltpu.sync_copy(posf_hbm, pf)
    pltpu.sync_copy(src_hbm.at[pl.ds(base_w, epw)], siall)
    pltpu.sync_copy(dst_hbm.at[pl.ds(base_w, epw)], diall)
    lanes = lax.iota(jnp.int32, L)

    def issue_gather(t, b):
        for j in range(2):
            pltpu.async_copy(
                feat_hbm.at[siall.at[pl.ds(t * STEP + j * HALF, HALF)]],
                frows.at[b, pl.ds(j * HALF, HALF)], sg[b])

    def wait_gather(b):
        pltpu.make_async_copy(feat_hbm.at[pl.ds(0, STEP)], frows.at[b],
                              sg[b]).wait()

    def issue_wout(t, b):
        base = base_w + t * STEP
        pltpu.async_copy(frows.at[b], fsrc_hbm.at[pl.ds(base, STEP)], sw[b])
        pltpu.async_copy(yv.at[b], y_hbm.at[pl.ds(base, STEP)], sw[b])

    def wait_wout(b):
        pltpu.make_async_copy(frows.at[b], fsrc_hbm.at[pl.ds(0, STEP)],
                              sw[b]).wait()
        pltpu.make_async_copy(yv.at[b], y_hbm.at[pl.ds(0, STEP)], sw[b]).wait()

    def compute_y(t, b):
        for j in range(2):
            for g in range(HGROUPS):
                off = t * STEP + j * HALF + g * L
                ivs = siall[pl.ds(off, L)]
                ivd = diall[pl.ds(off, L)]
                yx = plsc.load_gather(pf, [ivs]) - plsc.load_gather(pf, [ivd])
                ivs = ivs + N_NODES
                ivd = ivd + N_NODES
                yy = plsc.load_gather(pf, [ivs]) - plsc.load_gather(pf, [ivd])
                ivs = ivs + N_NODES
                ivd = ivd + N_NODES
                yz = plsc.load_gather(pf, [ivs]) - plsc.load_gather(pf, [ivd])
                rows = lanes + (j * HALF + g * L)
                plsc.store_scatter(yv.at[b],
                                   [rows, jnp.zeros((L,), jnp.int32)], yx)
                plsc.store_scatter(yv.at[b],
                                   [rows, jnp.ones((L,), jnp.int32)], yy)
                plsc.store_scatter(yv.at[b],
                                   [rows, jnp.full((L,), 2, jnp.int32)], yz)

    issue_gather(0, 0)

    def body(o, carry):
        for b in range(2):
            t = 2 * o + b
            compute_y(t, b)
            wait_gather(b)
            issue_wout(t, b)

            @pl.when(jnp.logical_and(t >= 1, t + 1 < nstep))
            def _():
                wait_wout(1 - b)

            @pl.when(t + 1 < nstep)
            def _():
                issue_gather(t + 1, 1 - b)
        return carry

    lax.fori_loop(0, nstep // 2, body, 0)
    wait_wout(0)
    wait_wout(1)

  return _gather_body


def _make_scatter_body(epw):
  chunks = epw // CHUNK

  def _scatter_body(dst_hbm, msg_hbm, zeros_hbm, out_hbm, didx, mrows, acc,
                    sl0, sl1, ss0, ss1):
    c = lax.axis_index("c")
    s = lax.axis_index("s")
    wid = s * NC + c
    base_w = wid * epw
    sl = (sl0, sl1)
    ss = (ss0, ss1)
    r0 = s * ROWS_PER_TILE
    pltpu.sync_copy(zeros_hbm.at[pl.ds(r0, ROWS_PER_TILE)],
                    acc.at[pl.ds(r0, ROWS_PER_TILE)])
    plsc.subcore_barrier()

    def issue_load(t, b):
        base = base_w + t * CHUNK
        pltpu.async_copy(dst_hbm.at[pl.ds(base, CHUNK)], didx.at[b], sl[b])
        pltpu.async_copy(msg_hbm.at[pl.ds(base, CHUNK)], mrows.at[b], sl[b])

    def wait_load(b):
        pltpu.make_async_copy(dst_hbm.at[pl.ds(0, CHUNK)], didx.at[b],
                              sl[b]).wait()
        pltpu.make_async_copy(msg_hbm.at[pl.ds(0, CHUNK)], mrows.at[b],
                              sl[b]).wait()

    def wait_scatter(b):
        pltpu.make_async_copy(msg_hbm.at[pl.ds(0, CHUNK)], mrows.at[b],
                              ss[b]).wait()

    issue_load(0, 0)

    def body(o, carry):
        for b in range(2):
            t = 2 * o + b
            wait_load(b)
            pltpu.async_copy(mrows.at[b], acc.at[didx.at[b]], ss[b], add=True)

            @pl.when(t >= 1)
            def _():
                wait_scatter(1 - b)

            @pl.when(t + 1 < chunks)
            def _():
                issue_load(t + 1, 1 - b)
        return carry

    lax.fori_loop(0, chunks // 2, body, 0)
    wait_scatter(1)
    plsc.subcore_barrier()
    pltpu.sync_copy(acc.at[pl.ds(r0, ROWS_PER_TILE)],
                    out_hbm.at[c, pl.ds(r0, ROWS_PER_TILE)])

  return _scatter_body


BB = 1024


def _msg_body(fsrc_ref, y_ref, kpt_ref, wall_ref, sel_ref, msg_ref):
    col = lax.broadcasted_iota(jnp.int32, (BB, PD), 1)
    y = jnp.where(col < 3, y_ref[...], 0.0)
    kpt = kpt_ref[...]
    cross = jnp.dot(y, kpt, preferred_element_type=jnp.float32)
    yn2 = jnp.sum(y * y, axis=1, keepdims=True)
    kn2 = jnp.sum(kpt * kpt, axis=0, keepdims=True)
    d2 = jnp.maximum(yn2 + kn2 - 2.0 * cross, 0.0) + 1e-12
    h = jnp.maximum(1.0 - jnp.sqrt(d2) * (1.0 / KP_EXTENT), 0.0)
    f16 = fsrc_ref[...].astype(jnp.bfloat16)
    t = jnp.dot(f16, wall_ref[...], preferred_element_type=jnp.float32)
    hb = jnp.dot(h.astype(jnp.bfloat16), sel_ref[...],
                 preferred_element_type=jnp.float32)
    acc = hb[:, 0:OUT_DIM] * t[:, 0:OUT_DIM]
    for k in range(1, KPAD):
        sl = slice(k * OUT_DIM, (k + 1) * OUT_DIM)
        acc = acc + hb[:, sl] * t[:, sl]
    msg_ref[...] = acc


def _add4_body(a_ref, b_ref, c_ref, d_ref, o_ref):
    o_ref[...] = (a_ref[...] + b_ref[...]) + (c_ref[...] + d_ref[...])


@jax.jit
def kernel(feat, pos, edge_index, weights, kernel_points):
    src = edge_index[0]
    dst = edge_index[1]
    epad = E_PAD - E_EDGES
    src_p = jnp.concatenate([src, jnp.zeros((epad,), jnp.int32)])
    dst_p = jnp.concatenate([dst, jnp.full((epad,), N_NODES, jnp.int32)])
    posf = jnp.concatenate([pos[:, 0], pos[:, 1], pos[:, 2]])
    kpt = jnp.pad(kernel_points, ((0, KPAD - K), (0, PD - kernel_points.shape[1]))).T
    w_all = jnp.transpose(
        jnp.pad(weights, ((0, KPAD - K), (0, 0), (0, 0))), (1, 0, 2)
    ).reshape(IN_DIM, KPAD * OUT_DIM).astype(jnp.bfloat16)
    sel = jnp.repeat(jnp.eye(KPAD, dtype=jnp.float32), OUT_DIM,
                     axis=1).astype(jnp.bfloat16)

    mesh = plsc.VectorSubcoreMesh(core_axis_name="c", subcore_axis_name="s")

    EH = E_PAD // 2
    ehw = EH // NW

    gather_fn = pl.kernel(
        _make_gather_body(ehw),
        out_type=[
            jax.ShapeDtypeStruct((EH, IN_DIM), jnp.float32),
            jax.ShapeDtypeStruct((EH, PD), jnp.float32),
        ],
        mesh=mesh,
        compiler_params=pltpu.CompilerParams(needs_layout_passes=False),
        scratch_types=[
            pltpu.VMEM((ehw,), jnp.int32),
            pltpu.VMEM((ehw,), jnp.int32),
            pltpu.VMEM((2, STEP, IN_DIM), jnp.float32),
            pltpu.VMEM((2, STEP, PD), jnp.float32),
            pltpu.VMEM((3 * N_NODES,), jnp.float32),
            pltpu.SemaphoreType.DMA,
            pltpu.SemaphoreType.DMA,
            pltpu.SemaphoreType.DMA,
            pltpu.SemaphoreType.DMA,
        ],
    )

    scatter_fn = pl.kernel(
        _make_scatter_body(ehw),
        out_type=jax.ShapeDtypeStruct((NC, N_PAD, OUT_DIM), jnp.float32),
        mesh=mesh,
        scratch_types=[
            pltpu.VMEM((2, CHUNK), jnp.int32),
            pltpu.VMEM((2, CHUNK, OUT_DIM), jnp.float32),
            pltpu.VMEM_SHARED((N_PAD, OUT_DIM), jnp.float32),
            pltpu.SemaphoreType.DMA,
            pltpu.SemaphoreType.DMA,
            pltpu.SemaphoreType.DMA,
            pltpu.SemaphoreType.DMA,
        ],
    )
    zeros_hbm = jnp.zeros((N_PAD, OUT_DIM), jnp.float32)

    msg_fn = pl.pallas_call(
        _msg_body,
        grid=(EH // BB,),
        in_specs=[
            pl.BlockSpec((BB, IN_DIM), lambda i: (i, 0)),
            pl.BlockSpec((BB, PD), lambda i: (i, 0)),
            pl.BlockSpec((PD, KPAD), lambda i: (0, 0)),
            pl.BlockSpec((IN_DIM, KPAD * OUT_DIM), lambda i: (0, 0)),
            pl.BlockSpec((KPAD, KPAD * OUT_DIM), lambda i: (0, 0)),
        ],
        out_specs=pl.BlockSpec((BB, OUT_DIM), lambda i: (i, 0)),
        out_shape=jax.ShapeDtypeStruct((EH, OUT_DIM), jnp.float32),
    )

    partials = []
    for h in range(2):
        src_h = lax.slice(src_p, (h * EH,), ((h + 1) * EH,))
        dst_h = lax.slice(dst_p, (h * EH,), ((h + 1) * EH,))
        fsrc, yarr = gather_fn(src_h, dst_h, feat, posf)
        msg = msg_fn(fsrc, yarr, kpt, w_all, sel)
        partials.append(scatter_fn(dst_h, msg, zeros_hbm))

    out = pl.pallas_call(
        _add4_body,
        grid=(10,),
        in_specs=[pl.BlockSpec((1000, OUT_DIM), lambda i: (i, 0))] * 4,
        out_specs=pl.BlockSpec((1000, OUT_DIM), lambda i: (i, 0)),
        out_shape=jax.ShapeDtypeStruct((N_NODES, OUT_DIM), jnp.float32),
    )(partials[0][0, :N_NODES], partials[0][1, :N_NODES],
      partials[1][0, :N_NODES], partials[1][1, :N_NODES])
    return out


# speedup vs baseline: 1.1686x; 1.0214x over previous
import functools

import jax
import jax.numpy as jnp
from jax import lax
from jax.experimental import pallas as pl
from jax.experimental.pallas import tpu as pltpu
from jax.experimental.pallas import tpu_sc as plsc

K = 15
KPAD = 16
PD = 8
IN_DIM = 128
OUT_DIM = 128
N_NODES = 10000
N_PAD = 10112
E_EDGES = 160000
E_PAD = 163840
KP_EXTENT = 1.2

NC = 2
NS = 16
NW = NC * NS
L = 16
CHUNK = 128
GROUPS = CHUNK // L
EPW = E_PAD // NW
CHUNKS = EPW // CHUNK
ROWS_PER_TILE = N_PAD // NS


STEP = 160
HALF = STEP // 2
NSTEP = EPW // STEP
HGROUPS = HALF // L


def _make_gather_body(epw):
  nstep = epw // STEP

  def _gather_body(src_hbm, dst_hbm, feat_hbm, posf_hbm,
                   fsrc_hbm, y_hbm,
                   siall, diall, frows, yv, pf,
                   sg0, sg1, sw0, sw1):
    c = lax.axis_index("c")
    s = lax.axis_index("s")
    wid = s * NC + c
    base_w = wid * epw
    sg = (sg0, sg1)
    sw = (sw0, sw1)
    pltpu.sync_copy(posf_hbm, pf)
    pltpu.sync_copy(src_hbm.at[pl.ds(base_w, epw)], siall)
    pltpu.sync_copy(dst_hbm.at[pl.ds(base_w, epw)], diall)
    lanes = lax.iota(jnp.int32, L)

    def issue_gather(t, b):
        for j in range(2):
            pltpu.async_copy(
                feat_hbm.at[siall.at[pl.ds(t * STEP + j * HALF, HALF)]],
                frows.at[b, pl.ds(j * HALF, HALF)], sg[b])

    def wait_gather(b):
        pltpu.make_async_copy(feat_hbm.at[pl.ds(0, STEP)], frows.at[b],
                              sg[b]).wait()

    def issue_wout(t, b):
        base = base_w + t * STEP
        pltpu.async_copy(frows.at[b], fsrc_hbm.at[pl.ds(base, STEP)], sw[b])
        pltpu.async_copy(yv.at[b], y_hbm.at[pl.ds(base, STEP)], sw[b])

    def wait_wout(b):
        pltpu.make_async_copy(frows.at[b], fsrc_hbm.at[pl.ds(0, STEP)],
                              sw[b]).wait()
        pltpu.make_async_copy(yv.at[b], y_hbm.at[pl.ds(0, STEP)], sw[b]).wait()

    def compute_y(t, b):
        for j in range(2):
            for g in range(HGROUPS):
                off = t * STEP + j * HALF + g * L
                ivs = siall[pl.ds(off, L)]
                ivd = diall[pl.ds(off, L)]
                yx = plsc.load_gather(pf, [ivs]) - plsc.load_gather(pf, [ivd])
                ivs = ivs + N_NODES
                ivd = ivd + N_NODES
                yy = plsc.load_gather(pf, [ivs]) - plsc.load_gather(pf, [ivd])
                ivs = ivs + N_NODES
                ivd = ivd + N_NODES
                yz = plsc.load_gather(pf, [ivs]) - plsc.load_gather(pf, [ivd])
                rows = lanes + (j * HALF + g * L)
                plsc.store_scatter(yv.at[b],
                                   [rows, jnp.zeros((L,), jnp.int32)], yx)
                plsc.store_scatter(yv.at[b],
                                   [rows, jnp.ones((L,), jnp.int32)], yy)
                plsc.store_scatter(yv.at[b],
                                   [rows, jnp.full((L,), 2, jnp.int32)], yz)

    issue_gather(0, 0)

    def body(o, carry):
        for b in range(2):
            t = 2 * o + b
            compute_y(t, b)
            wait_gather(b)
            issue_wout(t, b)

            @pl.when(jnp.logical_and(t >= 1, t + 1 < nstep))
            def _():
                wait_wout(1 - b)

            @pl.when(t + 1 < nstep)
            def _():
                issue_gather(t + 1, 1 - b)
        return carry

    lax.fori_loop(0, nstep // 2, body, 0)
    wait_wout(0)
    wait_wout(1)

  return _gather_body


def _make_scatter_body(epw):
  chunks = epw // CHUNK

  def _scatter_body(dst_hbm, msg_hbm, zeros_hbm, out_hbm, didx, mrows, acc,
                    sl0, sl1, ss0, ss1):
    c = lax.axis_index("c")
    s = lax.axis_index("s")
    wid = s * NC + c
    base_w = wid * epw
    sl = (sl0, sl1)
    ss = (ss0, ss1)
    r0 = s * ROWS_PER_TILE
    pltpu.sync_copy(zeros_hbm.at[pl.ds(r0, ROWS_PER_TILE)],
                    acc.at[pl.ds(r0, ROWS_PER_TILE)])
    plsc.subcore_barrier()

    def issue_load(t, b):
        base = base_w + t * CHUNK
        pltpu.async_copy(dst_hbm.at[pl.ds(base, CHUNK)], didx.at[b], sl[b])
        pltpu.async_copy(msg_hbm.at[pl.ds(base, CHUNK)], mrows.at[b], sl[b])

    def wait_load(b):
        pltpu.make_async_copy(dst_hbm.at[pl.ds(0, CHUNK)], didx.at[b],
                              sl[b]).wait()
        pltpu.make_async_copy(msg_hbm.at[pl.ds(0, CHUNK)], mrows.at[b],
                              sl[b]).wait()

    def wait_scatter(b):
        pltpu.make_async_copy(msg_hbm.at[pl.ds(0, CHUNK)], mrows.at[b],
                              ss[b]).wait()

    issue_load(0, 0)

    def body(o, carry):
        for b in range(2):
            t = 2 * o + b
            wait_load(b)
            pltpu.async_copy(mrows.at[b], acc.at[didx.at[b]], ss[b], add=True)

            @pl.when(t >= 1)
            def _():
                wait_scatter(1 - b)

            @pl.when(t + 1 < chunks)
            def _():
                issue_load(t + 1, 1 - b)
        return carry

    lax.fori_loop(0, chunks // 2, body, 0)
    wait_scatter(1)
    plsc.subcore_barrier()
    pltpu.sync_copy(acc.at[pl.ds(r0, ROWS_PER_TILE)],
                    out_hbm.at[c, pl.ds(r0, ROWS_PER_TILE)])

  return _scatter_body


BB = 1024


def _msg_body(fsrc_ref, y_ref, kpt_ref, wall_ref, sel_ref, msg_ref):
    col = lax.broadcasted_iota(jnp.int32, (BB, PD), 1)
    y = jnp.where(col < 3, y_ref[...], 0.0)
    kpt = kpt_ref[...]
    cross = jnp.dot(y, kpt, preferred_element_type=jnp.float32)
    yn2 = jnp.sum(y * y, axis=1, keepdims=True)
    kn2 = jnp.sum(kpt * kpt, axis=0, keepdims=True)
    d2 = jnp.maximum(yn2 + kn2 - 2.0 * cross, 0.0) + 1e-12
    h = jnp.maximum(1.0 - jnp.sqrt(d2) * (1.0 / KP_EXTENT), 0.0)
    f16 = fsrc_ref[...].astype(jnp.bfloat16)
    t = jnp.dot(f16, wall_ref[...], preferred_element_type=jnp.float32)
    hb = jnp.dot(h.astype(jnp.bfloat16), sel_ref[...],
                 preferred_element_type=jnp.float32)
    acc = hb[:, 0:OUT_DIM] * t[:, 0:OUT_DIM]
    for k in range(1, KPAD):
        sl = slice(k * OUT_DIM, (k + 1) * OUT_DIM)
        acc = acc + hb[:, sl] * t[:, sl]
    msg_ref[...] = acc


def _add4_body(a_ref, b_ref, c_ref, d_ref, o_ref):
    o_ref[...] = (a_ref[...] + b_ref[...]) + (c_ref[...] + d_ref[...])


def _add2_body(a_ref, b_ref, o_ref):
    o_ref[...] = a_ref[...] + b_ref[...]


@jax.jit
def kernel(feat, pos, edge_index, weights, kernel_points):
    src = edge_index[0]
    dst = edge_index[1]
    epad = E_PAD - E_EDGES
    src_p = jnp.concatenate([src, jnp.zeros((epad,), jnp.int32)])
    dst_p = jnp.concatenate([dst, jnp.full((epad,), N_NODES, jnp.int32)])
    posf = jnp.concatenate([pos[:, 0], pos[:, 1], pos[:, 2]])
    kpt = jnp.pad(kernel_points, ((0, KPAD - K), (0, PD - kernel_points.shape[1]))).T
    w_all = jnp.transpose(
        jnp.pad(weights, ((0, KPAD - K), (0, 0), (0, 0))), (1, 0, 2)
    ).reshape(IN_DIM, KPAD * OUT_DIM).astype(jnp.bfloat16)
    sel = jnp.repeat(jnp.eye(KPAD, dtype=jnp.float32), OUT_DIM,
                     axis=1).astype(jnp.bfloat16)

    mesh = plsc.VectorSubcoreMesh(core_axis_name="c", subcore_axis_name="s")

    NH = 4
    EH = E_PAD // NH
    ehw = EH // NW

    gather_fn = pl.kernel(
        _make_gather_body(ehw),
        out_type=[
            jax.ShapeDtypeStruct((EH, IN_DIM), jnp.float32),
            jax.ShapeDtypeStruct((EH, PD), jnp.float32),
        ],
        mesh=mesh,
        compiler_params=pltpu.CompilerParams(needs_layout_passes=False),
        scratch_types=[
            pltpu.VMEM((ehw,), jnp.int32),
            pltpu.VMEM((ehw,), jnp.int32),
            pltpu.VMEM((2, STEP, IN_DIM), jnp.float32),
            pltpu.VMEM((2, STEP, PD), jnp.float32),
            pltpu.VMEM((3 * N_NODES,), jnp.float32),
            pltpu.SemaphoreType.DMA,
            pltpu.SemaphoreType.DMA,
            pltpu.SemaphoreType.DMA,
            pltpu.SemaphoreType.DMA,
        ],
    )

    scatter_fn = pl.kernel(
        _make_scatter_body(ehw),
        out_type=jax.ShapeDtypeStruct((NC, N_PAD, OUT_DIM), jnp.float32),
        mesh=mesh,
        scratch_types=[
            pltpu.VMEM((2, CHUNK), jnp.int32),
            pltpu.VMEM((2, CHUNK, OUT_DIM), jnp.float32),
            pltpu.VMEM_SHARED((N_PAD, OUT_DIM), jnp.float32),
            pltpu.SemaphoreType.DMA,
            pltpu.SemaphoreType.DMA,
            pltpu.SemaphoreType.DMA,
            pltpu.SemaphoreType.DMA,
        ],
    )
    zeros_hbm = jnp.zeros((N_PAD, OUT_DIM), jnp.float32)

    msg_fn = pl.pallas_call(
        _msg_body,
        grid=(EH // BB,),
        in_specs=[
            pl.BlockSpec((BB, IN_DIM), lambda i: (i, 0)),
            pl.BlockSpec((BB, PD), lambda i: (i, 0)),
            pl.BlockSpec((PD, KPAD), lambda i: (0, 0)),
            pl.BlockSpec((IN_DIM, KPAD * OUT_DIM), lambda i: (0, 0)),
            pl.BlockSpec((KPAD, KPAD * OUT_DIM), lambda i: (0, 0)),
        ],
        out_specs=pl.BlockSpec((BB, OUT_DIM), lambda i: (i, 0)),
        out_shape=jax.ShapeDtypeStruct((EH, OUT_DIM), jnp.float32),
    )

    parts = []
    for h in range(NH):
        src_h = lax.slice(src_p, (h * EH,), ((h + 1) * EH,))
        dst_h = lax.slice(dst_p, (h * EH,), ((h + 1) * EH,))
        fsrc, yarr = gather_fn(src_h, dst_h, feat, posf)
        msg = msg_fn(fsrc, yarr, kpt, w_all, sel)
        p = scatter_fn(dst_h, msg, zeros_hbm)
        parts.append(p[0, :N_NODES])
        parts.append(p[1, :N_NODES])

    out = pl.pallas_call(
        _add4_body,
        grid=(10,),
        in_specs=[pl.BlockSpec((1000, OUT_DIM), lambda i: (i, 0))] * 4,
        out_specs=pl.BlockSpec((1000, OUT_DIM), lambda i: (i, 0)),
        out_shape=jax.ShapeDtypeStruct((N_NODES, OUT_DIM), jnp.float32),
    )(*parts[:4])
    if len(parts) > 4:
        out2 = pl.pallas_call(
            _add4_body,
            grid=(10,),
            in_specs=[pl.BlockSpec((1000, OUT_DIM), lambda i: (i, 0))] * 4,
            out_specs=pl.BlockSpec((1000, OUT_DIM), lambda i: (i, 0)),
            out_shape=jax.ShapeDtypeStruct((N_NODES, OUT_DIM), jnp.float32),
        )(*parts[4:8])
        out = pl.pallas_call(
            _add2_body,
            grid=(10,),
            in_specs=[pl.BlockSpec((1000, OUT_DIM), lambda i: (i, 0))] * 2,
            out_specs=pl.BlockSpec((1000, OUT_DIM), lambda i: (i, 0)),
            out_shape=jax.ShapeDtypeStruct((N_NODES, OUT_DIM), jnp.float32),
        )(out, out2)
    return out
